# baseline (speedup 1.0000x reference)
import functools

import jax
import jax.numpy as jnp
from jax import lax
from jax.experimental import pallas as pl
from jax.experimental.pallas import tpu as pltpu
from jax.experimental.pallas import tpu_sc as plsc

BATCH = 4096
SEQ = 200
D_MODEL = 64
BLK = 128
NUM_WORKERS = 32
NBUF = 4

N_ITEMS = (BATCH // BLK) * SEQ
ITEMS_PER_W = N_ITEMS // NUM_WORKERS
B_BLOCKS = BATCH // BLK


def _make_kernel():
    mesh = plsc.VectorSubcoreMesh(core_axis_name="c", subcore_axis_name="s")

    scratch = [pltpu.VMEM((ITEMS_PER_W, BLK), jnp.int32)]
    scratch += [pltpu.VMEM((BLK, D_MODEL), jnp.float32) for _ in range(NBUF)]
    scratch += [pltpu.VMEM((D_MODEL * BLK,), jnp.float32) for _ in range(NBUF)]
    scratch += [pltpu.SemaphoreType.DMA for _ in range(2 * NBUF)]

    @functools.partial(
        pl.kernel,
        mesh=mesh,
        compiler_params=pltpu.CompilerParams(
            use_tc_tiling_on_sc=False, needs_layout_passes=False
        ),
        out_type=jax.ShapeDtypeStruct((SEQ * 8 * B_BLOCKS, 8 * BLK), jnp.float32),
        scratch_types=scratch,
    )
    def gather_kernel(table_hbm, idx_hbm, out_hbm, idx_v, *rest):
        gbufs = rest[:NBUF]
        tbufs = rest[NBUF : 2 * NBUF]
        gsem = rest[2 * NBUF : 3 * NBUF]
        osem = rest[3 * NBUF :]
        wid = lax.axis_index("s") * 2 + lax.axis_index("c")
        item_base = wid * ITEMS_PER_W
        pltpu.sync_copy(idx_hbm.at[pl.ds(item_base, ITEMS_PER_W)], idx_v)

        addr_c = [jnp.arange(16, dtype=jnp.int32) * BLK + c * 16 * BLK for c in range(4)]

        def issue_gather(i, u):
            pltpu.async_copy(table_hbm.at[idx_v.at[i]], gbufs[u], gsem[u])

        def wait_gather(i, u):
            pltpu.make_async_copy(table_hbm.at[idx_v.at[i]], gbufs[u], gsem[u]).wait()

        def out_row(i, d_hi):
            j = item_base + i
            return (j // B_BLOCKS) * 256 + d_hi * 32 + (j % B_BLOCKS)

        def issue_outs(i, u):
            for d_hi in range(8):
                pltpu.async_copy(
                    tbufs[u].at[pl.ds(d_hi * 8 * BLK, 8 * BLK)],
                    out_hbm.at[out_row(i, d_hi)],
                    osem[u],
                )

        def wait_outs(i, u):
            for d_hi in range(8):
                pltpu.make_async_copy(
                    tbufs[u].at[pl.ds(d_hi * 8 * BLK, 8 * BLK)],
                    out_hbm.at[out_row(i, d_hi)],
                    osem[u],
                ).wait()

        def transpose(u):
            def body_b(b, addrs):
                for c in range(4):
                    v = gbufs[u][b, pl.ds(c * 16, 16)]
                    plsc.store_scatter(tbufs[u], [addrs[c]], v)
                return tuple(a + 1 for a in addrs)

            lax.fori_loop(0, BLK, body_b, tuple(addr_c), unroll=16)

        for u in range(NBUF - 1):
            issue_gather(u, u)

        def body(i, u, *, first_lap=False, last_lap=False):
            wait_gather(i, u)
            if not last_lap:
                issue_gather(i + NBUF - 1, (u + NBUF - 1) % NBUF)
            if not first_lap:
                wait_outs(i - NBUF, u)
            transpose(u)
            issue_outs(i, u)

        for u in range(NBUF):
            body(u, u, first_lap=True)

        def group(g, carry):
            i0 = g * NBUF
            for u in range(NBUF):
                body(i0 + u, u)
            return carry

        lax.fori_loop(1, ITEMS_PER_W // NBUF - 1, group, 0)

        i0 = ITEMS_PER_W - NBUF
        for u in range(NBUF):
            body(i0 + u, u, last_lap=u >= 1)

        for u in range(NBUF):
            wait_outs(i0 + u, u)

    return gather_kernel


_gather = _make_kernel()


@jax.jit
def kernel(x, table):
    idx2 = x.T.reshape(N_ITEMS, BLK).astype(jnp.int32)
    out2 = _gather(table, idx2)
    out5 = out2.reshape(SEQ, 8, B_BLOCKS, 8, BLK)
    return out5.transpose(2, 4, 0, 1, 3).reshape(BATCH, SEQ, D_MODEL)


# speedup vs baseline: 1.1812x; 1.1812x over previous
import functools

import jax
import jax.numpy as jnp
from jax import lax
from jax.experimental import pallas as pl
from jax.experimental.pallas import tpu as pltpu
from jax.experimental.pallas import tpu_sc as plsc

BATCH = 4096
SEQ = 200
D_MODEL = 64
BLK = 128
NUM_WORKERS = 32
NBUF = 4

N_ITEMS = (BATCH // BLK) * SEQ
ITEMS_PER_W = N_ITEMS // NUM_WORKERS
B_BLOCKS = BATCH // BLK


def _make_kernel():
    mesh = plsc.VectorSubcoreMesh(core_axis_name="c", subcore_axis_name="s")

    scratch = [pltpu.VMEM((ITEMS_PER_W, BLK), jnp.int32)]
    scratch += [pltpu.VMEM((BLK, D_MODEL), jnp.float32) for _ in range(NBUF)]
    scratch += [pltpu.VMEM((D_MODEL * BLK,), jnp.float32) for _ in range(NBUF)]
    scratch += [pltpu.SemaphoreType.DMA for _ in range(2 * NBUF)]

    @functools.partial(
        pl.kernel,
        mesh=mesh,
        compiler_params=pltpu.CompilerParams(
            use_tc_tiling_on_sc=False, needs_layout_passes=False
        ),
        out_type=jax.ShapeDtypeStruct((SEQ * 8 * B_BLOCKS, 8 * BLK), jnp.float32),
        scratch_types=scratch,
    )
    def gather_kernel(table_hbm, idx_hbm, out_hbm, idx_v, *rest):
        gbufs = rest[:NBUF]
        tbufs = rest[NBUF : 2 * NBUF]
        gsem = rest[2 * NBUF : 3 * NBUF]
        osem = rest[3 * NBUF :]
        wid = lax.axis_index("s") * 2 + lax.axis_index("c")
        item_base = wid * ITEMS_PER_W
        pltpu.sync_copy(idx_hbm.at[pl.ds(item_base, ITEMS_PER_W)], idx_v)

        addr_c = [jnp.arange(16, dtype=jnp.int32) * BLK + c * 16 * BLK for c in range(4)]

        def issue_gather(i, u):
            pltpu.async_copy(table_hbm.at[idx_v.at[i]], gbufs[u], gsem[u])

        def wait_gather(i, u):
            pltpu.make_async_copy(table_hbm.at[idx_v.at[i]], gbufs[u], gsem[u]).wait()

        def out_row(i, d_hi):
            j = item_base + i
            return (j // B_BLOCKS) * 256 + d_hi * 32 + (j % B_BLOCKS)

        def issue_outs(i, u):
            for d_hi in range(8):
                pltpu.async_copy(
                    tbufs[u].at[pl.ds(d_hi * 8 * BLK, 8 * BLK)],
                    out_hbm.at[out_row(i, d_hi)],
                    osem[u],
                )

        def wait_outs(i, u):
            for d_hi in range(8):
                pltpu.make_async_copy(
                    tbufs[u].at[pl.ds(d_hi * 8 * BLK, 8 * BLK)],
                    out_hbm.at[out_row(i, d_hi)],
                    osem[u],
                ).wait()

        def transpose(u):
            @plsc.parallel_loop(0, BLK, unroll=16, carry=tuple(addr_c))
            def body_b(b, addrs):
                for c in range(4):
                    v = gbufs[u][b, pl.ds(c * 16, 16)]
                    plsc.store_scatter(tbufs[u], [addrs[c]], v)
                return tuple(a + 1 for a in addrs)

        for u in range(NBUF - 1):
            issue_gather(u, u)

        def body(i, u, *, first_lap=False, last_lap=False):
            wait_gather(i, u)
            if not last_lap:
                issue_gather(i + NBUF - 1, (u + NBUF - 1) % NBUF)
            if not first_lap:
                wait_outs(i - NBUF, u)
            transpose(u)
            issue_outs(i, u)

        for u in range(NBUF):
            body(u, u, first_lap=True)

        def group(g, carry):
            i0 = g * NBUF
            for u in range(NBUF):
                body(i0 + u, u)
            return carry

        lax.fori_loop(1, ITEMS_PER_W // NBUF - 1, group, 0)

        i0 = ITEMS_PER_W - NBUF
        for u in range(NBUF):
            body(i0 + u, u, last_lap=u >= 1)

        for u in range(NBUF):
            wait_outs(i0 + u, u)

    return gather_kernel


_gather = _make_kernel()


@jax.jit
def kernel(x, table):
    idx2 = x.T.reshape(N_ITEMS, BLK).astype(jnp.int32)
    out2 = _gather(table, idx2)
    out5 = out2.reshape(SEQ, 8, B_BLOCKS, 8, BLK)
    return out5.transpose(2, 4, 0, 1, 3).reshape(BATCH, SEQ, D_MODEL)


# speedup vs baseline: 1.2346x; 1.0452x over previous
import functools

import jax
import jax.numpy as jnp
from jax import lax
from jax.experimental import pallas as pl
from jax.experimental.pallas import tpu as pltpu
from jax.experimental.pallas import tpu_sc as plsc

BATCH = 4096
SEQ = 200
D_MODEL = 64
BLK = 128
NUM_WORKERS = 32
NBUF = 4

N_ITEMS = (BATCH // BLK) * SEQ
ITEMS_PER_W = N_ITEMS // NUM_WORKERS
B_BLOCKS = BATCH // BLK


def _make_kernel():
    mesh = plsc.VectorSubcoreMesh(core_axis_name="c", subcore_axis_name="s")

    scratch = [pltpu.VMEM((ITEMS_PER_W, BLK), jnp.int32)]
    scratch += [pltpu.VMEM((BLK, D_MODEL), jnp.float32) for _ in range(NBUF)]
    scratch += [pltpu.VMEM((D_MODEL * BLK,), jnp.float32) for _ in range(NBUF)]
    scratch += [pltpu.SemaphoreType.DMA for _ in range(2 * NBUF)]

    @functools.partial(
        pl.kernel,
        mesh=mesh,
        compiler_params=pltpu.CompilerParams(
            use_tc_tiling_on_sc=False, needs_layout_passes=False
        ),
        out_type=jax.ShapeDtypeStruct((SEQ * 8 * B_BLOCKS, 8 * BLK), jnp.float32),
        scratch_types=scratch,
    )
    def gather_kernel(table_hbm, idx_hbm, out_hbm, idx_v, *rest):
        gbufs = rest[:NBUF]
        tbufs = rest[NBUF : 2 * NBUF]
        gsem = rest[2 * NBUF : 3 * NBUF]
        osem = rest[3 * NBUF :]
        wid = lax.axis_index("s") * 2 + lax.axis_index("c")
        item_base = wid * ITEMS_PER_W
        pltpu.sync_copy(idx_hbm.at[pl.ds(item_base, ITEMS_PER_W)], idx_v)

        rows_k = [jnp.arange(16, dtype=jnp.int32) + k * 16 for k in range(8)]

        def issue_gather(i, u):
            pltpu.async_copy(table_hbm.at[idx_v.at[i]], gbufs[u], gsem[u])

        def wait_gather(i, u):
            pltpu.make_async_copy(table_hbm.at[idx_v.at[i]], gbufs[u], gsem[u]).wait()

        def out_row(i, d_hi):
            j = item_base + i
            return (j // B_BLOCKS) * 256 + d_hi * 32 + (j % B_BLOCKS)

        def issue_outs(i, u):
            for d_hi in range(8):
                pltpu.async_copy(
                    tbufs[u].at[pl.ds(d_hi * 8 * BLK, 8 * BLK)],
                    out_hbm.at[out_row(i, d_hi)],
                    osem[u],
                )

        def wait_outs(i, u):
            for d_hi in range(8):
                pltpu.make_async_copy(
                    tbufs[u].at[pl.ds(d_hi * 8 * BLK, 8 * BLK)],
                    out_hbm.at[out_row(i, d_hi)],
                    osem[u],
                ).wait()

        def transpose(u):
            @plsc.parallel_loop(0, D_MODEL, unroll=8)
            def body_d(d):
                cols = jnp.full((16,), 0, jnp.int32) + d
                for k in range(8):
                    v = plsc.load_gather(gbufs[u], [rows_k[k], cols])
                    tbufs[u][pl.ds(d * BLK + k * 16, 16)] = v

        for u in range(NBUF - 1):
            issue_gather(u, u)

        def body(i, u, *, first_lap=False, last_lap=False):
            wait_gather(i, u)
            if not last_lap:
                issue_gather(i + NBUF - 1, (u + NBUF - 1) % NBUF)
            if not first_lap:
                wait_outs(i - NBUF, u)
            transpose(u)
            issue_outs(i, u)

        for u in range(NBUF):
            body(u, u, first_lap=True)

        def group(g, carry):
            i0 = g * NBUF
            for u in range(NBUF):
                body(i0 + u, u)
            return carry

        lax.fori_loop(1, ITEMS_PER_W // NBUF - 1, group, 0)

        i0 = ITEMS_PER_W - NBUF
        for u in range(NBUF):
            body(i0 + u, u, last_lap=u >= 1)

        for u in range(NBUF):
            wait_outs(i0 + u, u)

    return gather_kernel


_gather = _make_kernel()


@jax.jit
def kernel(x, table):
    idx2 = x.T.reshape(N_ITEMS, BLK).astype(jnp.int32)
    out2 = _gather(table, idx2)
    out5 = out2.reshape(SEQ, 8, B_BLOCKS, 8, BLK)
    return out5.transpose(2, 4, 0, 1, 3).reshape(BATCH, SEQ, D_MODEL)


# speedup vs baseline: 2.1170x; 1.7147x over previous
import functools

import jax
import jax.numpy as jnp
from jax import lax
from jax.experimental import pallas as pl
from jax.experimental.pallas import tpu as pltpu
from jax.experimental.pallas import tpu_sc as plsc

BATCH = 4096
SEQ = 200
D_MODEL = 64
BLK = 128
NUM_WORKERS = 32
NBUF = 4

N_ITEMS = (BATCH // BLK) * SEQ
ITEMS_PER_W = N_ITEMS // NUM_WORKERS
B_BLOCKS = BATCH // BLK


def _make_kernel():
    mesh = plsc.VectorSubcoreMesh(core_axis_name="c", subcore_axis_name="s")

    scratch = [pltpu.VMEM((ITEMS_PER_W, BLK), jnp.int32)]
    scratch += [pltpu.VMEM((BLK, D_MODEL), jnp.float32) for _ in range(NBUF)]
    scratch += [pltpu.VMEM((D_MODEL * BLK,), jnp.float32) for _ in range(NBUF)]
    scratch += [pltpu.SemaphoreType.DMA for _ in range(2 * NBUF)]

    @functools.partial(
        pl.kernel,
        mesh=mesh,
        compiler_params=pltpu.CompilerParams(
            use_tc_tiling_on_sc=False, needs_layout_passes=False
        ),
        out_type=jax.ShapeDtypeStruct((SEQ * 8 * B_BLOCKS, 8 * BLK), jnp.float32),
        scratch_types=scratch,
    )
    def gather_kernel(table_hbm, idx_hbm, out_hbm, idx_v, *rest):
        gbufs = rest[:NBUF]
        tbufs = rest[NBUF : 2 * NBUF]
        gsem = rest[2 * NBUF : 3 * NBUF]
        osem = rest[3 * NBUF :]
        wid = lax.axis_index("s") * 2 + lax.axis_index("c")
        item_base = wid * ITEMS_PER_W
        pltpu.sync_copy(idx_hbm.at[pl.ds(item_base, ITEMS_PER_W)], idx_v)

        lane = jnp.arange(16, dtype=jnp.int32)
        cdiag = [(lane + j) % 16 for j in range(16)]
        sdiag = [((lane + j) % 16) * BLK + lane for j in range(16)]

        def issue_gather(i, u):
            pltpu.async_copy(table_hbm.at[idx_v.at[i]], gbufs[u], gsem[u])

        def wait_gather(i, u):
            pltpu.make_async_copy(table_hbm.at[idx_v.at[i]], gbufs[u], gsem[u]).wait()

        def out_row(i, d_hi):
            j = item_base + i
            return (j // B_BLOCKS) * 256 + d_hi * 32 + (j % B_BLOCKS)

        def issue_outs(i, u):
            for d_hi in range(8):
                pltpu.async_copy(
                    tbufs[u].at[pl.ds(d_hi * 8 * BLK, 8 * BLK)],
                    out_hbm.at[out_row(i, d_hi)],
                    osem[u],
                )

        def wait_outs(i, u):
            for d_hi in range(8):
                pltpu.make_async_copy(
                    tbufs[u].at[pl.ds(d_hi * 8 * BLK, 8 * BLK)],
                    out_hbm.at[out_row(i, d_hi)],
                    osem[u],
                ).wait()

        def transpose(u):
            @plsc.parallel_loop(0, 32)
            def body_blk(i):
                b0 = (i >> 2) * 16
                c = i & 3
                rows = lane + b0
                coff = c * 16
                soff = c * 2048 + b0
                for j in range(16):
                    v = plsc.load_gather(gbufs[u], [rows, cdiag[j] + coff])
                    plsc.store_scatter(tbufs[u], [sdiag[j] + soff], v)

        for u in range(NBUF - 1):
            issue_gather(u, u)

        def body(i, u, *, first_lap=False, last_lap=False):
            wait_gather(i, u)
            if not last_lap:
                issue_gather(i + NBUF - 1, (u + NBUF - 1) % NBUF)
            if not first_lap:
                wait_outs(i - NBUF, u)
            transpose(u)
            issue_outs(i, u)

        for u in range(NBUF):
            body(u, u, first_lap=True)

        def group(g, carry):
            i0 = g * NBUF
            for u in range(NBUF):
                body(i0 + u, u)
            return carry

        lax.fori_loop(1, ITEMS_PER_W // NBUF - 1, group, 0)

        i0 = ITEMS_PER_W - NBUF
        for u in range(NBUF):
            body(i0 + u, u, last_lap=u >= 1)

        for u in range(NBUF):
            wait_outs(i0 + u, u)

    return gather_kernel


_gather = _make_kernel()


@jax.jit
def kernel(x, table):
    idx2 = x.T.reshape(N_ITEMS, BLK).astype(jnp.int32)
    out2 = _gather(table, idx2)
    out5 = out2.reshape(SEQ, 8, B_BLOCKS, 8, BLK)
    return out5.transpose(2, 4, 0, 1, 3).reshape(BATCH, SEQ, D_MODEL)
